# baseline (speedup 1.0000x reference)
import functools

import numpy as np

import jax
import jax.numpy as jnp
from jax import lax
from jax.experimental import pallas as pl
from jax.experimental.pallas import tpu as pltpu
from jax.experimental.pallas import tpu_sc as plsc

EMBED = 64
NUM_CLASSES = 32
BATCH = 4096
SEQ = 200

NC = 2
NS = 16
NW = NC * NS
BPW = BATCH // NW
CHUNK = 100
NCHUNK = SEQ // CHUNK
LANES = 16
NVREG = EMBED // LANES


VOCAB = 1000000
PANEL = 4096
NPANEL = VOCAB // PANEL
NB = NPANEL // 4
TAIL0 = NPANEL * PANEL
TAILC = VOCAB - TAIL0


def _tc_transpose(table_t, tail_packed):
    C = PANEL

    def body(t_ref, tail_ref, o_ref):
        i = pl.program_id(0)

        @pl.when(i < NB)
        def _():
            t = t_ref[...]
            for q in range(4):
                tq = t[:, q * C : (q + 1) * C]
                o_ref[pl.ds(q * (C // 2), C // 2), :] = jnp.concatenate(
                    [tq[:, 0 : C // 2].T, tq[:, C // 2 :].T], axis=1
                ).astype(jnp.bfloat16)

        @pl.when(i == NB)
        def _():
            o_ref[pl.ds(0, TAILC // 2), :] = tail_ref[...]

    return pl.pallas_call(
        body,
        grid=(NB + 1,),
        in_specs=[
            pl.BlockSpec((EMBED, 4 * C), lambda i: (0, jnp.minimum(i, NB - 1))),
            pl.BlockSpec((TAILC // 2, 2 * EMBED), lambda i: (0, 0)),
        ],
        out_specs=pl.BlockSpec(
            (2 * C, 2 * EMBED), lambda i: (jnp.minimum(i, NB), 0)
        ),
        out_shape=jax.ShapeDtypeStruct((VOCAB // 2, 2 * EMBED), jnp.bfloat16),
    )(table_t, tail_packed)


def _sc_pool(x_r, table):
    mesh = plsc.VectorSubcoreMesh(core_axis_name="c", subcore_axis_name="s")

    @functools.partial(
        pl.kernel,
        out_type=jax.ShapeDtypeStruct((NW, BPW, EMBED), jnp.float32),
        mesh=mesh,
        scratch_types=[
            pltpu.VMEM((BPW, NCHUNK, CHUNK), jnp.int32),
            pltpu.VMEM((2, SEQ, EMBED), jnp.bfloat16),
            pltpu.VMEM((BPW, EMBED), jnp.float32),
            pltpu.SemaphoreType.DMA,
            pltpu.SemaphoreType.DMA,
        ],
        compiler_params=pltpu.CompilerParams(
            use_tc_tiling_on_sc=False, needs_layout_passes=False
        ),
    )
    def pool(x_hbm, table_hbm, out_hbm, idx_v, buf_v, acc_v, sem0, sem1):
        wid = lax.axis_index("s") * NC + lax.axis_index("c")
        sems = (sem0, sem1)
        pltpu.sync_copy(x_hbm.at[wid], idx_v)

        def issue(b, p):
            for j in range(NCHUNK):
                pltpu.async_copy(
                    table_hbm.at[idx_v.at[b, j]],
                    buf_v.at[p, pl.ds(j * CHUNK, CHUNK)],
                    sems[p],
                )

        def drain(p):
            pltpu.make_async_copy(
                table_hbm.at[pl.ds(0, SEQ)], buf_v.at[p], sems[p]
            ).wait()

        issue(0, 0)
        issue(1, 1)

        def outer(g, carry):
            for p in range(2):
                b = g * 2 + p
                drain(p)

                @pl.when(b + 2 < BPW)
                def _():
                    issue(b + 2, p)

                zero = jnp.zeros((LANES,), jnp.float32)

                def inner(i, accs):
                    out = list(accs)
                    for u in range(4):
                        r = i * 4 + u
                        s = (u % 2) * NVREG
                        for h in range(2):
                            v = buf_v[p, r, pl.ds(2 * LANES * h, 2 * LANES)]
                            ea, ob = plsc.unpack(
                                v, format=plsc.PackFormat.INTERLEAVED
                            )
                            out[s + 2 * h] = out[s + 2 * h] + ea
                            out[s + 2 * h + 1] = out[s + 2 * h + 1] + ob
                    return tuple(out)

                accs = lax.fori_loop(0, SEQ // 4, inner, (zero,) * (2 * NVREG))
                for k in range(NVREG):
                    acc_v[b, pl.ds(LANES * k, LANES)] = accs[k] + accs[NVREG + k]
            return carry

        lax.fori_loop(0, BPW // 2, outer, 0)
        pltpu.sync_copy(acc_v, out_hbm.at[wid])

    return pool(x_r, table)


def _tc_head(sums, wt, bias):
    def head(s_ref, w_ref, b_ref, o_ref):
        doc = s_ref[...] * (1.0 / SEQ)
        o_ref[...] = (
            jnp.dot(doc, w_ref[...], preferred_element_type=jnp.float32) + b_ref[...]
        )

    return pl.pallas_call(
        head,
        out_shape=jax.ShapeDtypeStruct((BATCH, NUM_CLASSES), jnp.float32),
    )(sums, wt, bias)


def _permute_idx(x):
    blk = x & ~(PANEL - 1)
    r = x & (PANEL - 1)
    perm = blk + 2 * (r & (PANEL // 2 - 1)) + (r >> 11)
    return jnp.where(x >= TAIL0, x, perm)


_EPERM = np.concatenate([
    np.arange(0, 2 * LANES, 2), np.arange(1, 2 * LANES, 2),
    2 * LANES + np.arange(0, 2 * LANES, 2), 2 * LANES + np.arange(1, 2 * LANES, 2),
])


def kernel(x, table, W, b):
    x_r = _permute_idx(x.astype(jnp.int32)).reshape(NW, BPW, NCHUNK, CHUNK)
    tail_packed = table[TAIL0:].astype(jnp.bfloat16).reshape(TAILC // 2, 2 * EMBED)
    table_packed = _tc_transpose(table.T, tail_packed)
    table_rm = table_packed.reshape(VOCAB, EMBED)
    sums = _sc_pool(x_r, table_rm)
    wt_perm = W.T[_EPERM]
    return _tc_head(sums.reshape(BATCH, EMBED), wt_perm, b.reshape(1, NUM_CLASSES))


# speedup vs baseline: 1.8536x; 1.8536x over previous
import functools

import jax
import jax.numpy as jnp
from jax import lax
from jax.experimental import pallas as pl
from jax.experimental.pallas import tpu as pltpu
from jax.experimental.pallas import tpu_sc as plsc

EMBED = 64
NUM_CLASSES = 32
BATCH = 4096
SEQ = 200

NC = 2
NS = 16
NW = NC * NS
BPW = BATCH // NW
CHUNK = 100
NCHUNK = SEQ // CHUNK
LANES = 16
NVREG = EMBED // LANES


VOCAB = 1000000
PANEL = 4096
NPANEL = VOCAB // PANEL
NB = NPANEL // 4
TAIL0 = NPANEL * PANEL
TAILC = VOCAB - TAIL0


def _tc_transpose(table_t, tail_packed):
    C = PANEL

    def body(t_ref, tail_ref, o_ref):
        i = pl.program_id(0)

        @pl.when(i < NB)
        def _():
            t = t_ref[...]
            for q in range(4):
                tq = t[:, q * C : (q + 1) * C]
                o_ref[pl.ds(q * (C // 2), C // 2), :] = jnp.concatenate(
                    [tq[:, 0 : C // 2].T, tq[:, C // 2 :].T], axis=1
                )

        @pl.when(i == NB)
        def _():
            o_ref[pl.ds(0, TAILC // 2), :] = tail_ref[...]

    return pl.pallas_call(
        body,
        grid=(NB + 1,),
        in_specs=[
            pl.BlockSpec((EMBED, 4 * C), lambda i: (0, jnp.minimum(i, NB - 1))),
            pl.BlockSpec((TAILC // 2, 2 * EMBED), lambda i: (0, 0)),
        ],
        out_specs=pl.BlockSpec(
            (2 * C, 2 * EMBED), lambda i: (jnp.minimum(i, NB), 0)
        ),
        out_shape=jax.ShapeDtypeStruct((VOCAB // 2, 2 * EMBED), jnp.float32),
    )(table_t, tail_packed)


def _sc_pool(x_r, table):
    mesh = plsc.VectorSubcoreMesh(core_axis_name="c", subcore_axis_name="s")

    @functools.partial(
        pl.kernel,
        out_type=jax.ShapeDtypeStruct((NW, BPW, EMBED), jnp.float32),
        mesh=mesh,
        scratch_types=[
            pltpu.VMEM((BPW, NCHUNK, CHUNK), jnp.int32),
            pltpu.VMEM((2, SEQ, EMBED), jnp.float32),
            pltpu.VMEM((BPW, EMBED), jnp.float32),
            pltpu.SemaphoreType.DMA,
            pltpu.SemaphoreType.DMA,
        ],
        compiler_params=pltpu.CompilerParams(use_tc_tiling_on_sc=False),
    )
    def pool(x_hbm, table_hbm, out_hbm, idx_v, buf_v, acc_v, sem0, sem1):
        wid = lax.axis_index("s") * NC + lax.axis_index("c")
        sems = (sem0, sem1)
        pltpu.sync_copy(x_hbm.at[wid], idx_v)

        def issue(b, p):
            for j in range(NCHUNK):
                pltpu.async_copy(
                    table_hbm.at[idx_v.at[b, j]],
                    buf_v.at[p, pl.ds(j * CHUNK, CHUNK)],
                    sems[p],
                )

        def drain(p):
            pltpu.make_async_copy(
                table_hbm.at[pl.ds(0, SEQ)], buf_v.at[p], sems[p]
            ).wait()

        issue(0, 0)
        issue(1, 1)

        def outer(g, carry):
            for p in range(2):
                b = g * 2 + p
                drain(p)

                @pl.when(b + 2 < BPW)
                def _():
                    issue(b + 2, p)

                zero = jnp.zeros((LANES,), jnp.float32)

                def inner(i, accs):
                    out = list(accs)
                    for u in range(4):
                        r = i * 4 + u
                        s = (u % 2) * NVREG
                        for k in range(NVREG):
                            out[s + k] = out[s + k] + buf_v[p, r, pl.ds(LANES * k, LANES)]
                    return tuple(out)

                accs = lax.fori_loop(0, SEQ // 4, inner, (zero,) * (2 * NVREG))
                for k in range(NVREG):
                    acc_v[b, pl.ds(LANES * k, LANES)] = accs[k] + accs[NVREG + k]
            return carry

        lax.fori_loop(0, BPW // 2, outer, 0)
        pltpu.sync_copy(acc_v, out_hbm.at[wid])

    return pool(x_r, table)


def _tc_head(sums, wt, bias):
    def head(s_ref, w_ref, b_ref, o_ref):
        doc = s_ref[...] * (1.0 / SEQ)
        o_ref[...] = (
            jnp.dot(doc, w_ref[...], preferred_element_type=jnp.float32) + b_ref[...]
        )

    return pl.pallas_call(
        head,
        out_shape=jax.ShapeDtypeStruct((BATCH, NUM_CLASSES), jnp.float32),
    )(sums, wt, bias)


def _permute_idx(x):
    blk = x & ~(PANEL - 1)
    r = x & (PANEL - 1)
    perm = blk + 2 * (r & (PANEL // 2 - 1)) + (r >> 11)
    return jnp.where(x >= TAIL0, x, perm)


def kernel(x, table, W, b):
    x_r = _permute_idx(x.astype(jnp.int32)).reshape(NW, BPW, NCHUNK, CHUNK)
    tail_packed = table[TAIL0:].reshape(TAILC // 2, 2 * EMBED)
    table_packed = _tc_transpose(table.T, tail_packed)
    table_rm = table_packed.reshape(VOCAB, EMBED)
    sums = _sc_pool(x_r, table_rm)
    return _tc_head(sums.reshape(BATCH, EMBED), W.T, b.reshape(1, NUM_CLASSES))
